# baseline (speedup 1.0000x reference)
import functools

import jax
import jax.numpy as jnp
from jax import lax
from jax.experimental import pallas as pl
from jax.experimental.pallas import tpu as pltpu
from jax.experimental.pallas import tpu_sc as plsc

_NC = 2
_NS = 16
_NW = _NC * _NS
_L = 16
_G = 8


@functools.partial(jax.jit, static_argnums=(4, 5, 6, 7))
def _gather_mul(idx, idxg, wgrp, aux, B, F, H, C):
  R = B * F
  GW = _G * H
  cpf = B // C
  nch = R // (C * _NW)
  mesh = plsc.VectorSubcoreMesh(core_axis_name="c", subcore_axis_name="s")

  @functools.partial(
      pl.kernel,
      out_type=jax.ShapeDtypeStruct((F * H * B,), jnp.float32),
      mesh=mesh,
      scratch_types=[
          [pltpu.VMEM((C,), jnp.int32)] * 2,
          [pltpu.VMEM((C,), jnp.int32)] * 2,
          [pltpu.VMEM((C, GW), jnp.float32)] * 2,
          [pltpu.VMEM((C,), jnp.int32)] * 2,
          [pltpu.VMEM((H * C,), jnp.float32)] * 2,
          [pltpu.SemaphoreType.DMA] * 2,
          [pltpu.SemaphoreType.DMA] * 2,
          [pltpu.SemaphoreType.DMA] * 2,
      ],
      compiler_params=pltpu.CompilerParams(
          needs_layout_passes=False, use_tc_tiling_on_sc=True
      ),
  )
  def gk(idx_hbm, idxg_hbm, w_hbm, aux_hbm, out_hbm, idx_v, idxg_v, w_v,
         m_v, col_v, sem_w, sem_m, sem_o):
    wid = lax.axis_index("s") * _NC + lax.axis_index("c")
    lanes = lax.iota(jnp.int32, _L)
    lane_base = lanes * C
    q0 = wid * nch

    def start_gathers(c, p):
      pltpu.sync_copy(idx_hbm.at[pl.ds((q0 + c) * C, C)], idx_v[p])
      pltpu.sync_copy(idxg_hbm.at[pl.ds((q0 + c) * C, C)], idxg_v[p])
      pltpu.async_copy(w_hbm.at[idxg_v[p]], w_v[p], sem_w[p])
      pltpu.async_copy(aux_hbm.at[idx_v[p]], m_v[p], sem_m[p])

    def wait_gathers(p):
      pltpu.make_async_copy(w_hbm.at[idxg_v[p]], w_v[p], sem_w[p]).wait()
      pltpu.make_async_copy(aux_hbm.at[idx_v[p]], m_v[p], sem_m[p]).wait()

    def drain_out(p):
      for h in range(H):
        pltpu.make_async_copy(
            col_v[p].at[pl.ds(h * C, C)],
            out_hbm.at[pl.ds(h * B, C)],
            sem_o[p],
        ).wait()

    def compute(c, p):
      @plsc.parallel_loop(0, C, 1, unroll=4)
      def _row(j):
        aux = plsc.load_gather(m_v[p], [jnp.full((_L,), j, jnp.int32)])
        sub = ((aux >> 16) & (_G - 1)) << 4
        val = plsc.load_gather(
            w_v[p], [jnp.full((_L,), j, jnp.int32), sub + lanes]
        )
        bits = (aux >> lanes) & 1
        plsc.store_scatter(col_v[p], [lane_base + j],
                           val * bits.astype(jnp.float32))

      q = q0 + c
      f = q // cpf
      b0 = (q % cpf) * C
      obase = f * (H * B) + b0
      for h in range(H):
        pltpu.async_copy(
            col_v[p].at[pl.ds(h * C, C)],
            out_hbm.at[pl.ds(obase + h * B, C)],
            sem_o[p],
        )

    assert nch % 2 == 0 and nch >= 4
    start_gathers(0, 0)
    start_gathers(1, 1)
    wait_gathers(0)
    compute(0, 0)
    start_gathers(2, 0)
    wait_gathers(1)
    compute(1, 1)
    start_gathers(3, 1)

    def pair(i, carry):
      cA = 2 * i
      wait_gathers(0)
      drain_out(0)
      compute(cA, 0)
      start_gathers(cA + 2, 0)
      wait_gathers(1)
      drain_out(1)
      compute(cA + 1, 1)
      start_gathers(cA + 3, 1)
      return carry

    lax.fori_loop(1, nch // 2 - 1, pair, 0)

    wait_gathers(0)
    drain_out(0)
    compute(nch - 2, 0)
    wait_gathers(1)
    drain_out(1)
    compute(nch - 1, 1)
    drain_out(0)
    drain_out(1)

  return gk(idx, idxg, wgrp, aux)


def kernel(x, weight, mask):
  B, F = x.shape
  V, H = weight.shape
  idx = jnp.swapaxes(x, 0, 1).reshape(B * F).astype(jnp.int32)
  idxg = idx >> 3
  pow2 = jnp.asarray([float(1 << i) for i in range(H)], dtype=jnp.float32)
  words = jnp.dot(mask.astype(jnp.float32), pow2).astype(jnp.int32)
  aux = words | ((jnp.arange(V, dtype=jnp.int32) & 7) << 16)
  wgrp = weight.reshape(V // _G, _G * H)
  flat = _gather_mul(idx, idxg, wgrp, aux, B, F, H, 256)
  return flat.reshape(F, H, B).transpose(2, 0, 1)
